# Initial kernel scaffold; baseline (speedup 1.0000x reference)
import jax
import jax.numpy as jnp
from jax.experimental import pallas as pl


def kernel(x, Wq, Wk, Wv):
    raise NotImplementedError("write your pallas kernel here")


# speedup vs baseline: 2.2504x; 2.2504x over previous
import jax
import jax.numpy as jnp
from jax.experimental import pallas as pl

B, N, C = 4, 4096, 1024
M = 1024
QB = 256
SCALE = 1.0 / 32.0
ND = C + 128


def _qkv_body(x_ref, w_ref, q_ref, k_ref, v_ref):
    x = x_ref[0]
    q_ref[0] = jnp.dot(x, w_ref[0], preferred_element_type=jnp.float32)
    k_ref[0] = jnp.dot(x, w_ref[1], preferred_element_type=jnp.float32)
    v_ref[0] = jnp.dot(x, w_ref[2], preferred_element_type=jnp.float32)


def _attn_body(q_ref, k_ref, v_ref, nd_ref):
    q = q_ref[0]
    k = k_ref[0]
    v = v_ref[0]
    s = jax.lax.dot_general(q, k, (((1,), (1,)), ((), ())),
                            preferred_element_type=jnp.float32) * SCALE
    e = jnp.exp(s)
    den = jnp.sum(e, axis=1, keepdims=True)
    num = jnp.dot(e, v, preferred_element_type=jnp.float32)
    nd_ref[0, :, :C] = num
    nd_ref[0, :, C:] = jnp.broadcast_to(den, (QB, 128))


def _up2(a):
    r, w = a.shape
    return jnp.stack([a, jnp.zeros_like(a)], axis=1).reshape(2 * r, w)


def _up4(a):
    r, w = a.shape
    z = jnp.zeros((r, 3, w), dtype=a.dtype)
    return jnp.concatenate([a[:, None, :], z], axis=1).reshape(4 * r, w)


def _combine_body(q_ref, k_ref, v_ref, nd1_ref, nd2_ref, out_ref):
    q = q_ref[0]
    k = k_ref[0]
    v = v_ref[0]
    s = jax.lax.dot_general(q, k, (((1,), (1,)), ((), ())),
                            preferred_element_type=jnp.float32) * SCALE
    e = jnp.exp(s)
    den = jnp.sum(e, axis=1, keepdims=True)
    num = jnp.dot(e, v, preferred_element_type=jnp.float32)
    nd1 = nd1_ref[0]
    nd2 = nd2_ref[0]
    num = num + _up2(nd1[:, :C]) + _up4(nd2[:, :C])
    den = den + _up2(nd1[:, C:C + 1]) + _up4(nd2[:, C:C + 1])
    out_ref[0] = num / den


def kernel(x, Wq, Wk, Wv):
    w = jnp.stack([Wq, Wk, Wv])

    bn = 256
    q, k, v = pl.pallas_call(
        _qkv_body,
        grid=(B, N // bn),
        in_specs=[
            pl.BlockSpec((1, bn, C), lambda b, i: (b, i, 0)),
            pl.BlockSpec((3, C, C), lambda b, i: (0, 0, 0)),
        ],
        out_specs=[pl.BlockSpec((1, bn, C), lambda b, i: (b, i, 0))] * 3,
        out_shape=[jax.ShapeDtypeStruct((B, N, C), jnp.float32)] * 3,
    )(x, w)

    def level(r):
        segs = 4 // r
        qv = q.reshape(B, N // r, r * C)
        kv = k.reshape(B, N // r, r * C)
        vv = v.reshape(B, N // r, r * C)
        tq = M // QB
        return pl.pallas_call(
            _attn_body,
            grid=(B, segs, tq),
            in_specs=[
                pl.BlockSpec((1, QB, C), lambda b, s, t: (b, s * tq + t, 0)),
                pl.BlockSpec((1, M, C), lambda b, s, t: (b, s, 0)),
                pl.BlockSpec((1, M, C), lambda b, s, t: (b, s, 0)),
            ],
            out_specs=pl.BlockSpec((1, QB, ND), lambda b, s, t: (b, s * tq + t, 0)),
            out_shape=jax.ShapeDtypeStruct((B, segs * M, ND), jnp.float32),
        )(qv, kv, vv)

    nd1 = level(2)
    nd2 = level(4)

    tq = M // QB
    out = pl.pallas_call(
        _combine_body,
        grid=(B, 4, tq),
        in_specs=[
            pl.BlockSpec((1, QB, C), lambda b, s, t: (b, s * tq + t, 0)),
            pl.BlockSpec((1, M, C), lambda b, s, t: (b, s, 0)),
            pl.BlockSpec((1, M, C), lambda b, s, t: (b, s, 0)),
            pl.BlockSpec((1, QB // 2, ND), lambda b, s, t: (b, s * tq + t, 0)),
            pl.BlockSpec((1, QB // 4, ND), lambda b, s, t: (b, s * tq + t, 0)),
        ],
        out_specs=pl.BlockSpec((1, QB, C), lambda b, s, t: (b, s * tq + t, 0)),
        out_shape=jax.ShapeDtypeStruct((B, N, C), jnp.float32),
    )(q, k, v, nd1, nd2)
    return out


# speedup vs baseline: 2.5501x; 1.1332x over previous
import jax
import jax.numpy as jnp
from jax.experimental import pallas as pl

B, N, C = 4, 4096, 1024
M = 1024
QB = 256
SCALE = 1.0 / 32.0
ND = C + 128


def _qkv_body(x_ref, w_ref, q_ref, k_ref, v_ref):
    x = x_ref[0]
    q_ref[0] = jnp.dot(x, w_ref[0],
                       preferred_element_type=jnp.float32).astype(jnp.bfloat16)
    k_ref[0] = jnp.dot(x, w_ref[1],
                       preferred_element_type=jnp.float32).astype(jnp.bfloat16)
    v_ref[0, :, :C] = jnp.dot(x, w_ref[2],
                              preferred_element_type=jnp.float32).astype(jnp.bfloat16)
    v_ref[0, :, C:] = jnp.ones((x.shape[0], 128), jnp.bfloat16)


def _attn_body(q_ref, k_ref, v_ref, nd_ref):
    q = q_ref[0]
    k = k_ref[0]
    v = v_ref[0]
    s = jax.lax.dot_general(q, k, (((1,), (1,)), ((), ())),
                            preferred_element_type=jnp.float32) * SCALE
    e = jnp.exp(s).astype(jnp.bfloat16)
    nd_ref[0] = jnp.dot(e, v, preferred_element_type=jnp.float32)


def _up2(a):
    r, w = a.shape
    return jnp.stack([a, jnp.zeros_like(a)], axis=1).reshape(2 * r, w)


def _up4(a):
    r, w = a.shape
    z = jnp.zeros((r, 3, w), dtype=a.dtype)
    return jnp.concatenate([a[:, None, :], z], axis=1).reshape(4 * r, w)


def _combine_body(q_ref, k_ref, v_ref, nd1_ref, nd2_ref, out_ref):
    q = q_ref[0]
    k = k_ref[0]
    v = v_ref[0]
    s = jax.lax.dot_general(q, k, (((1,), (1,)), ((), ())),
                            preferred_element_type=jnp.float32) * SCALE
    e = jnp.exp(s).astype(jnp.bfloat16)
    ne = jnp.dot(e, v, preferred_element_type=jnp.float32)
    nd1 = nd1_ref[0]
    nd2 = nd2_ref[0]
    num = ne[:, :C] + _up2(nd1[:, :C]) + _up4(nd2[:, :C])
    den = ne[:, C:C + 1] + _up2(nd1[:, C:C + 1]) + _up4(nd2[:, C:C + 1])
    out_ref[0] = num / den


def kernel(x, Wq, Wk, Wv):
    w = jnp.stack([Wq, Wk, Wv]).astype(jnp.bfloat16)
    xb = x.astype(jnp.bfloat16)

    bn = 256
    q, k, v = pl.pallas_call(
        _qkv_body,
        grid=(B, N // bn),
        in_specs=[
            pl.BlockSpec((1, bn, C), lambda b, i: (b, i, 0)),
            pl.BlockSpec((3, C, C), lambda b, i: (0, 0, 0)),
        ],
        out_specs=[
            pl.BlockSpec((1, bn, C), lambda b, i: (b, i, 0)),
            pl.BlockSpec((1, bn, C), lambda b, i: (b, i, 0)),
            pl.BlockSpec((1, bn, ND), lambda b, i: (b, i, 0)),
        ],
        out_shape=[
            jax.ShapeDtypeStruct((B, N, C), jnp.bfloat16),
            jax.ShapeDtypeStruct((B, N, C), jnp.bfloat16),
            jax.ShapeDtypeStruct((B, N, ND), jnp.bfloat16),
        ],
    )(xb, w)

    def level(r):
        segs = 4 // r
        qv = q.reshape(B, N // r, r * C)
        kv = k.reshape(B, N // r, r * C)
        vv = v.reshape(B, N // r, r * ND)
        tq = M // QB
        return pl.pallas_call(
            _attn_body,
            grid=(B, segs, tq),
            in_specs=[
                pl.BlockSpec((1, QB, C), lambda b, s, t: (b, s * tq + t, 0)),
                pl.BlockSpec((1, M, C), lambda b, s, t: (b, s, 0)),
                pl.BlockSpec((1, M, ND), lambda b, s, t: (b, s, 0)),
            ],
            out_specs=pl.BlockSpec((1, QB, ND), lambda b, s, t: (b, s * tq + t, 0)),
            out_shape=jax.ShapeDtypeStruct((B, segs * M, ND), jnp.float32),
        )(qv, kv, vv)

    nd1 = level(2)
    nd2 = level(4)

    tq = M // QB
    out = pl.pallas_call(
        _combine_body,
        grid=(B, 4, tq),
        in_specs=[
            pl.BlockSpec((1, QB, C), lambda b, s, t: (b, s * tq + t, 0)),
            pl.BlockSpec((1, M, C), lambda b, s, t: (b, s, 0)),
            pl.BlockSpec((1, M, ND), lambda b, s, t: (b, s, 0)),
            pl.BlockSpec((1, QB // 2, ND), lambda b, s, t: (b, s * tq + t, 0)),
            pl.BlockSpec((1, QB // 4, ND), lambda b, s, t: (b, s * tq + t, 0)),
        ],
        out_specs=pl.BlockSpec((1, QB, C), lambda b, s, t: (b, s * tq + t, 0)),
        out_shape=jax.ShapeDtypeStruct((B, N, C), jnp.float32),
    )(q, k, v, nd1, nd2)
    return out
